# Initial kernel scaffold; baseline (speedup 1.0000x reference)
import jax
import jax.numpy as jnp
from jax.experimental import pallas as pl


def kernel(x, edge_index, num_node, edge_weight, W, b):
    raise NotImplementedError("write your pallas kernel here")


# speedup vs baseline: 5.2777x; 5.2777x over previous
import functools

import jax
import jax.numpy as jnp
from jax import lax
from jax.experimental import pallas as pl
from jax.experimental.pallas import tpu as pltpu
from jax.experimental.pallas import tpu_sc as plsc

NC = 2
NS = 16
NW = NC * NS
L = 16

CHUNK = 80


def _linear_relu(x, W, b):
    n, d = x.shape
    blk = 1000
    assert n % blk == 0

    def body(x_ref, w_ref, b_ref, o_ref):
        y = lax.dot_general(x_ref[...], w_ref[...], (((1,), (1,)), ((), ())),
                            preferred_element_type=jnp.float32)
        o_ref[...] = jnp.maximum(y + b_ref[...], 0.0)

    return pl.pallas_call(
        body,
        grid=(n // blk,),
        in_specs=[
            pl.BlockSpec((blk, d), lambda i: (i, 0)),
            pl.BlockSpec((d, d), lambda i: (0, 0)),
            pl.BlockSpec((1, d), lambda i: (0, 0)),
        ],
        out_specs=pl.BlockSpec((blk, d), lambda i: (i, 0)),
        out_shape=jax.ShapeDtypeStruct((n, d), jnp.float32),
    )(x, W, b.reshape(1, d))


def _make_sc_agg(n, d, e):
    ept = e // NW
    assert ept * NW == e and ept % CHUNK == 0
    steps = ept // CHUNK
    span = -(-(n - 8 * CHUNK) // (NS - 1)) if NS > 1 else 0
    span -= span % 8
    zchunks = 8
    assert span * (NS - 1) + zchunks * CHUNK >= n and span > 0
    mesh = plsc.VectorSubcoreMesh(core_axis_name="c", subcore_axis_name="s",
                                  num_cores=NC, num_subcores=NS)

    @functools.partial(
        pl.kernel,
        out_type=(
            jax.ShapeDtypeStruct((NC, n, d), jnp.float32),
            jax.ShapeDtypeStruct((NW * n,), jnp.float32),
        ),
        mesh=mesh,
        scratch_types=[
            pltpu.VMEM_SHARED((n, d), jnp.float32),
            pltpu.VMEM((n,), jnp.float32),
            pltpu.VMEM((CHUNK,), jnp.int32),
            pltpu.VMEM((1, CHUNK), jnp.int32),
            pltpu.VMEM((CHUNK,), jnp.float32),
            pltpu.VMEM((CHUNK, d), jnp.float32),
            pltpu.SemaphoreType.DMA,
        ],
        compiler_params=pltpu.CompilerParams(needs_layout_passes=False),
    )
    def sc_agg(xt, src_h, dst_h, w_h, out, rs_out,
               aggr_sh, rs_v, dst_v, src_v, w_v, rows_v, sem):
        cid = lax.axis_index("c")
        sid = lax.axis_index("s")
        wid = sid * NC + cid

        zero16 = jnp.zeros((L,), jnp.float32)

        def zrow(i, _):
            rows_v[i // (d // L), pl.ds((i % (d // L)) * L, L)] = zero16
            return 0
        lax.fori_loop(0, CHUNK * (d // L), zrow, 0)

        def zrs(i, _):
            rs_v[pl.ds(i * L, L)] = zero16
            return 0
        lax.fori_loop(0, n // L, zrs, 0)

        row0 = pl.multiple_of(sid * span, 8)
        for k in range(zchunks):
            pltpu.sync_copy(rows_v, aggr_sh.at[pl.ds(row0 + k * CHUNK, CHUNK)])
        plsc.subcore_barrier()

        def chunk_body(t, _):
            base = pl.multiple_of(wid * ept + t * CHUNK, 8)
            c1 = pltpu.async_copy(dst_h.at[pl.ds(base, CHUNK)], dst_v, sem)
            c2 = pltpu.async_copy(src_h.at[pl.ds(base, CHUNK)], src_v.at[0], sem)
            c3 = pltpu.async_copy(w_h.at[pl.ds(base, CHUNK)], w_v, sem)
            c1.wait()
            c2.wait()
            c3.wait()
            pltpu.async_copy(xt.at[dst_v], rows_v, sem).wait()

            def row_body(i, _):
                wb = plsc.load_gather(w_v, [jnp.zeros((L,), jnp.int32) + i])
                for cb in range(d // L):
                    rows_v[i, pl.ds(cb * L, L)] = rows_v[i, pl.ds(cb * L, L)] * wb
                return 0
            lax.fori_loop(0, CHUNK, row_body, 0)

            def deg_body(g, _):
                sv = src_v[0, pl.ds(g * L, L)]
                wv = w_v[pl.ds(g * L, L)]
                plsc.addupdate_scatter(rs_v, [sv], wv)
                return 0
            lax.fori_loop(0, CHUNK // L, deg_body, 0)

            pltpu.sync_copy(rows_v, aggr_sh.at[src_v.at[0]], add=True)
            return 0

        lax.fori_loop(0, steps, chunk_body, 0)
        plsc.subcore_barrier()

        for k in range(zchunks):
            r = pl.multiple_of(row0 + k * CHUNK, 8)
            pltpu.sync_copy(aggr_sh.at[pl.ds(r, CHUNK)], rows_v)
            pltpu.sync_copy(rows_v, out.at[cid, pl.ds(r, CHUNK)])
        pltpu.sync_copy(rs_v, rs_out.at[pl.ds(wid * n, n)])

    return sc_agg


def _combine(parts, rs, n, d):
    blk = 1000
    assert n % blk == 0

    def body(p_ref, r_ref, o_ref):
        s = p_ref[0] + p_ref[1]
        deg = lax.dot_general(r_ref[0], jnp.ones((NW, 1), jnp.float32),
                              (((0,), (0,)), ((), ())),
                              preferred_element_type=jnp.float32)
        o_ref[...] = s / jnp.maximum(deg, 1.0)

    return pl.pallas_call(
        body,
        grid=(n // blk,),
        in_specs=[
            pl.BlockSpec((NC, blk, d), lambda i: (0, i, 0)),
            pl.BlockSpec((1, NW, blk), lambda i: (i, 0, 0)),
        ],
        out_specs=pl.BlockSpec((blk, d), lambda i: (i, 0)),
        out_shape=jax.ShapeDtypeStruct((n, d), jnp.float32),
    )(parts, rs)


def kernel(x, edge_index, num_node, edge_weight, W, b):
    del num_node
    n, d = x.shape
    e = edge_index.shape[1]
    xt = _linear_relu(x, W, b)
    src = edge_index[0]
    dst = edge_index[1]
    parts, rs = _make_sc_agg(n, d, e)(xt, src, dst, edge_weight)
    blk = 1000
    rs3 = rs.reshape(NW, n // blk, blk).swapaxes(0, 1)
    return _combine(parts, rs3, n, d)


# speedup vs baseline: 10.8312x; 2.0523x over previous
import functools

import jax
import jax.numpy as jnp
from jax import lax
from jax.experimental import pallas as pl
from jax.experimental.pallas import tpu as pltpu
from jax.experimental.pallas import tpu_sc as plsc

NC = 2
NS = 16
NW = NC * NS
L = 16

CHUNK = 80
NSLOT = 3
NROT = 3


def _linear_relu(x, W, b):
    n, d = x.shape
    blk = 1000
    assert n % blk == 0

    def body(x_ref, w_ref, b_ref, o_ref):
        y = lax.dot_general(x_ref[...], w_ref[...], (((1,), (1,)), ((), ())),
                            preferred_element_type=jnp.float32)
        o_ref[...] = jnp.maximum(y + b_ref[...], 0.0)

    return pl.pallas_call(
        body,
        grid=(n // blk,),
        in_specs=[
            pl.BlockSpec((blk, d), lambda i: (i, 0)),
            pl.BlockSpec((d, d), lambda i: (0, 0)),
            pl.BlockSpec((1, d), lambda i: (0, 0)),
        ],
        out_specs=pl.BlockSpec((blk, d), lambda i: (i, 0)),
        out_shape=jax.ShapeDtypeStruct((n, d), jnp.float32),
    )(x, W, b.reshape(1, d))


def _make_sc_agg(n, d, e):
    ept = e // NW
    assert ept * NW == e and ept % CHUNK == 0
    steps = ept // CHUNK
    trips = (steps - 2) // NSLOT
    assert trips * NSLOT + 2 == steps
    span = -(-(n - 8 * CHUNK) // (NS - 1)) if NS > 1 else 0
    span -= span % 8
    zchunks = 8
    assert span * (NS - 1) + zchunks * CHUNK >= n and span > 0
    mesh = plsc.VectorSubcoreMesh(core_axis_name="c", subcore_axis_name="s",
                                  num_cores=NC, num_subcores=NS)

    @functools.partial(
        pl.kernel,
        out_type=(
            jax.ShapeDtypeStruct((NC, n, d), jnp.float32),
            jax.ShapeDtypeStruct((NW * n,), jnp.float32),
        ),
        mesh=mesh,
        scratch_types=[
            pltpu.VMEM_SHARED((n, d), jnp.float32),
            pltpu.VMEM((n,), jnp.float32),
            pltpu.VMEM((NSLOT, CHUNK), jnp.int32),
            pltpu.VMEM((NSLOT, CHUNK), jnp.int32),
            pltpu.VMEM((NSLOT, CHUNK), jnp.float32),
            pltpu.VMEM((NROT, CHUNK, d), jnp.float32),
            pltpu.SemaphoreType.DMA,
            pltpu.SemaphoreType.DMA,
            pltpu.SemaphoreType.DMA,
        ],
        compiler_params=pltpu.CompilerParams(needs_layout_passes=False),
    )
    def sc_agg(xt, src_h, dst_h, w_h, out, rs_out,
               aggr_sh, rs_v, dst_r, src_r, w_r, rows_v, gsem, ssem, isem):
        cid = lax.axis_index("c")
        sid = lax.axis_index("s")
        wid = sid * NC + cid

        zero16 = jnp.zeros((L,), jnp.float32)

        def zrow(i, _):
            rows_v[i // (d // L) // CHUNK, (i // (d // L)) % CHUNK,
                   pl.ds((i % (d // L)) * L, L)] = zero16
            return 0
        lax.fori_loop(0, NROT * CHUNK * (d // L), zrow, 0)

        def zrs(i, _):
            rs_v[pl.ds(i * L, L)] = zero16
            return 0
        lax.fori_loop(0, n // L, zrs, 0)

        row0 = pl.multiple_of(sid * span, 8)
        for k in range(zchunks):
            pltpu.sync_copy(rows_v.at[0], aggr_sh.at[pl.ds(row0 + k * CHUNK, CHUNK)])
        plsc.subcore_barrier()

        ebase = pl.multiple_of(wid * ept, 8)

        def chunk_off(t):
            return pl.multiple_of(ebase + lax.rem(t, steps) * CHUNK, 8)

        def load_idx(t, slot):
            o = chunk_off(t)
            pltpu.async_copy(dst_h.at[pl.ds(o, CHUNK)], dst_r.at[slot], isem)
            pltpu.async_copy(src_h.at[pl.ds(o, CHUNK)], src_r.at[slot], isem)
            pltpu.async_copy(w_h.at[pl.ds(o, CHUNK)], w_r.at[slot], isem)

        def wait_idx(slot):
            for ref in (dst_r, src_r, w_r):
                pltpu.make_async_copy(dst_h.at[pl.ds(ebase, CHUNK)],
                                      ref.at[slot], isem).wait()

        def start_gather(slot, r):
            pltpu.async_copy(xt.at[dst_r.at[slot]], rows_v.at[r], gsem)

        def wait_gather(slot, r):
            pltpu.make_async_copy(xt.at[dst_r.at[slot]], rows_v.at[r],
                                  gsem).wait()

        def start_scatter(slot, r):
            pltpu.async_copy(rows_v.at[r], aggr_sh.at[src_r.at[slot]], ssem,
                             add=True)

        def wait_scatter(slot, r):
            pltpu.make_async_copy(rows_v.at[r], aggr_sh.at[src_r.at[slot]],
                                  ssem).wait()

        def scale_and_deg(slot, r):
            def row_body(i, _):
                for u in range(2):
                    row = i * 2 + u
                    wb = plsc.load_gather(w_r.at[slot],
                                          [jnp.zeros((L,), jnp.int32) + row])
                    for cb in range(d // L):
                        rows_v[r, row, pl.ds(cb * L, L)] = (
                            rows_v[r, row, pl.ds(cb * L, L)] * wb)
                return 0
            lax.fori_loop(0, CHUNK // 2, row_body, 0)

            for g in range(CHUNK // L):
                sv = src_r[slot, pl.ds(g * L, L)]
                wv = w_r[slot, pl.ds(g * L, L)]
                plsc.addupdate_scatter(rs_v, [sv], wv)

        load_idx(0, 0)
        for _ in range(3):
            pltpu.make_async_copy(dst_h.at[pl.ds(ebase, CHUNK)],
                                  dst_r.at[0], isem).wait()
        load_idx(1, 1)
        start_gather(0, 0)
        start_scatter(0, 1)
        start_scatter(0, 2)

        def trip_body(k, _):
            t0 = k * NSLOT
            for j in range(NSLOT):
                t = t0 + j
                sj = j
                sn = (j + 1) % NSLOT
                sp = (j + 2) % NSLOT
                wait_scatter(sn, sn)
                wait_idx(sn)
                start_gather(sn, sn)
                wait_gather(sj, sj)
                load_idx(t + 2, sp)
                scale_and_deg(sj, sj)
                start_scatter(sj, sj)
            return 0

        lax.fori_loop(0, trips, trip_body, 0)

        wait_scatter(1, 1)
        wait_idx(1)
        start_gather(1, 1)
        wait_gather(0, 0)
        scale_and_deg(0, 0)
        start_scatter(0, 0)
        wait_scatter(2, 2)
        wait_gather(1, 1)
        scale_and_deg(1, 1)
        start_scatter(1, 1)
        wait_scatter(0, 0)
        wait_scatter(1, 1)

        plsc.subcore_barrier()

        for k in range(zchunks):
            r = pl.multiple_of(row0 + k * CHUNK, 8)
            pltpu.sync_copy(aggr_sh.at[pl.ds(r, CHUNK)], rows_v.at[0])
            pltpu.sync_copy(rows_v.at[0], out.at[cid, pl.ds(r, CHUNK)])
        pltpu.sync_copy(rs_v, rs_out.at[pl.ds(wid * n, n)])

    return sc_agg


def _combine(parts, rs, n, d):
    blk = 1000
    assert n % blk == 0

    def body(p_ref, r_ref, o_ref):
        s = p_ref[0] + p_ref[1]
        deg = lax.dot_general(r_ref[0], jnp.ones((NW, 1), jnp.float32),
                              (((0,), (0,)), ((), ())),
                              preferred_element_type=jnp.float32)
        o_ref[...] = s / jnp.maximum(deg, 1.0)

    return pl.pallas_call(
        body,
        grid=(n // blk,),
        in_specs=[
            pl.BlockSpec((NC, blk, d), lambda i: (0, i, 0)),
            pl.BlockSpec((1, NW, blk), lambda i: (i, 0, 0)),
        ],
        out_specs=pl.BlockSpec((blk, d), lambda i: (i, 0)),
        out_shape=jax.ShapeDtypeStruct((n, d), jnp.float32),
    )(parts, rs)


def kernel(x, edge_index, num_node, edge_weight, W, b):
    del num_node
    n, d = x.shape
    e = edge_index.shape[1]
    xt = _linear_relu(x, W, b)
    src = edge_index[0]
    dst = edge_index[1]
    parts, rs = _make_sc_agg(n, d, e)(xt, src, dst, edge_weight)
    blk = 1000
    rs3 = rs.reshape(NW, n // blk, blk).swapaxes(0, 1)
    return _combine(parts, rs3, n, d)


# speedup vs baseline: 11.2524x; 1.0389x over previous
import functools

import jax
import jax.numpy as jnp
from jax import lax
from jax.experimental import pallas as pl
from jax.experimental.pallas import tpu as pltpu
from jax.experimental.pallas import tpu_sc as plsc

NC = 2
NS = 16
NW = NC * NS
L = 16

CHUNK = 80
NSLOT = 3
NROT = 3


def _linear_relu(x, W, b):
    n, d = x.shape
    blk = 1000
    assert n % blk == 0

    def body(x_ref, w_ref, b_ref, o_ref):
        y = lax.dot_general(x_ref[...], w_ref[...], (((1,), (1,)), ((), ())),
                            preferred_element_type=jnp.float32)
        o_ref[...] = jnp.maximum(y + b_ref[...], 0.0)

    return pl.pallas_call(
        body,
        grid=(n // blk,),
        in_specs=[
            pl.BlockSpec((blk, d), lambda i: (i, 0)),
            pl.BlockSpec((d, d), lambda i: (0, 0)),
            pl.BlockSpec((1, d), lambda i: (0, 0)),
        ],
        out_specs=pl.BlockSpec((blk, d), lambda i: (i, 0)),
        out_shape=jax.ShapeDtypeStruct((n, d), jnp.float32),
    )(x, W, b.reshape(1, d))


def _make_sc_agg(n, d, e):
    ept = e // NW
    assert ept * NW == e and ept % CHUNK == 0
    steps = ept // CHUNK
    trips = (steps - 2) // NSLOT
    assert trips * NSLOT + 2 == steps
    span = -(-(n - 8 * CHUNK) // (NS - 1)) if NS > 1 else 0
    span -= span % 8
    zchunks = 8
    assert span * (NS - 1) + zchunks * CHUNK >= n and span > 0
    mesh = plsc.VectorSubcoreMesh(core_axis_name="c", subcore_axis_name="s",
                                  num_cores=NC, num_subcores=NS)

    @functools.partial(
        pl.kernel,
        out_type=(
            jax.ShapeDtypeStruct((NC, n, d), jnp.float32),
            jax.ShapeDtypeStruct((NW * n,), jnp.float32),
        ),
        mesh=mesh,
        scratch_types=[
            pltpu.VMEM_SHARED((n, d), jnp.float32),
            pltpu.VMEM((n,), jnp.float32),
            pltpu.VMEM((NSLOT, CHUNK), jnp.int32),
            pltpu.VMEM((NSLOT, CHUNK), jnp.int32),
            pltpu.VMEM((NSLOT, CHUNK), jnp.float32),
            pltpu.VMEM((NROT, CHUNK, d), jnp.float32),
            pltpu.SemaphoreType.DMA,
            pltpu.SemaphoreType.DMA,
            pltpu.SemaphoreType.DMA,
        ],
        compiler_params=pltpu.CompilerParams(needs_layout_passes=False),
    )
    def sc_agg(xt, src_h, dst_h, w_h, out, rs_out,
               aggr_sh, rs_v, dst_r, src_r, w_r, rows_v, gsem, ssem, isem):
        cid = lax.axis_index("c")
        sid = lax.axis_index("s")
        wid = sid * NC + cid

        zero16 = jnp.zeros((L,), jnp.float32)

        def zrow(i, _):
            rows_v[i // (d // L) // CHUNK, (i // (d // L)) % CHUNK,
                   pl.ds((i % (d // L)) * L, L)] = zero16
            return 0
        lax.fori_loop(0, NROT * CHUNK * (d // L), zrow, 0)

        def zrs(i, _):
            rs_v[pl.ds(i * L, L)] = zero16
            return 0
        lax.fori_loop(0, n // L, zrs, 0)

        row0 = pl.multiple_of(sid * span, 8)
        for k in range(zchunks):
            pltpu.sync_copy(rows_v.at[0], aggr_sh.at[pl.ds(row0 + k * CHUNK, CHUNK)])
        plsc.subcore_barrier()

        ebase = pl.multiple_of(wid * ept, 8)

        def chunk_off(t):
            return pl.multiple_of(ebase + lax.rem(t, steps) * CHUNK, 8)

        def load_idx(t, slot):
            o = chunk_off(t)
            pltpu.async_copy(dst_h.at[pl.ds(o, CHUNK)], dst_r.at[slot], isem)
            pltpu.async_copy(src_h.at[pl.ds(o, CHUNK)], src_r.at[slot], isem)
            pltpu.async_copy(w_h.at[pl.ds(o, CHUNK)], w_r.at[slot], isem)

        def wait_idx(slot):
            for ref in (dst_r, src_r, w_r):
                pltpu.make_async_copy(dst_h.at[pl.ds(ebase, CHUNK)],
                                      ref.at[slot], isem).wait()

        def start_gather(slot, r):
            pltpu.async_copy(xt.at[dst_r.at[slot]], rows_v.at[r], gsem)

        def wait_gather(slot, r):
            pltpu.make_async_copy(xt.at[dst_r.at[slot]], rows_v.at[r],
                                  gsem).wait()

        def start_scatter(slot, r):
            pltpu.async_copy(rows_v.at[r], aggr_sh.at[src_r.at[slot]], ssem,
                             add=True)

        def wait_scatter(slot, r):
            pltpu.make_async_copy(rows_v.at[r], aggr_sh.at[src_r.at[slot]],
                                  ssem).wait()

        def scale_and_deg(slot, r):
            @plsc.parallel_loop(0, CHUNK, unroll=4)
            def _row_body(i):
                wb = plsc.load_gather(w_r.at[slot],
                                      [jnp.zeros((L,), jnp.int32) + i])
                for cb in range(d // L):
                    rows_v[r, i, pl.ds(cb * L, L)] = (
                        rows_v[r, i, pl.ds(cb * L, L)] * wb)

            for g in range(CHUNK // L):
                sv = src_r[slot, pl.ds(g * L, L)]
                wv = w_r[slot, pl.ds(g * L, L)]
                plsc.addupdate_scatter(rs_v, [sv], wv)

        load_idx(0, 0)
        for _ in range(3):
            pltpu.make_async_copy(dst_h.at[pl.ds(ebase, CHUNK)],
                                  dst_r.at[0], isem).wait()
        load_idx(1, 1)
        start_gather(0, 0)
        start_scatter(0, 1)
        start_scatter(0, 2)

        def trip_body(k, _):
            t0 = k * NSLOT
            for j in range(NSLOT):
                t = t0 + j
                sj = j
                sn = (j + 1) % NSLOT
                sp = (j + 2) % NSLOT
                wait_scatter(sn, sn)
                wait_idx(sn)
                start_gather(sn, sn)
                wait_gather(sj, sj)
                load_idx(t + 2, sp)
                scale_and_deg(sj, sj)
                start_scatter(sj, sj)
            return 0

        lax.fori_loop(0, trips, trip_body, 0)

        wait_scatter(1, 1)
        wait_idx(1)
        start_gather(1, 1)
        wait_gather(0, 0)
        scale_and_deg(0, 0)
        start_scatter(0, 0)
        wait_scatter(2, 2)
        wait_gather(1, 1)
        scale_and_deg(1, 1)
        start_scatter(1, 1)
        wait_scatter(0, 0)
        wait_scatter(1, 1)

        plsc.subcore_barrier()

        for k in range(zchunks):
            r = pl.multiple_of(row0 + k * CHUNK, 8)
            pltpu.sync_copy(aggr_sh.at[pl.ds(r, CHUNK)], rows_v.at[0])
            pltpu.sync_copy(rows_v.at[0], out.at[cid, pl.ds(r, CHUNK)])
        pltpu.sync_copy(rs_v, rs_out.at[pl.ds(wid * n, n)])

    return sc_agg


def _combine(parts, rs, n, d):
    blk = 1000
    assert n % blk == 0

    def body(p_ref, r_ref, o_ref):
        s = p_ref[0] + p_ref[1]
        deg = lax.dot_general(r_ref[0], jnp.ones((NW, 1), jnp.float32),
                              (((0,), (0,)), ((), ())),
                              preferred_element_type=jnp.float32)
        o_ref[...] = s / jnp.maximum(deg, 1.0)

    return pl.pallas_call(
        body,
        grid=(n // blk,),
        in_specs=[
            pl.BlockSpec((NC, blk, d), lambda i: (0, i, 0)),
            pl.BlockSpec((1, NW, blk), lambda i: (i, 0, 0)),
        ],
        out_specs=pl.BlockSpec((blk, d), lambda i: (i, 0)),
        out_shape=jax.ShapeDtypeStruct((n, d), jnp.float32),
    )(parts, rs)


def kernel(x, edge_index, num_node, edge_weight, W, b):
    del num_node
    n, d = x.shape
    e = edge_index.shape[1]
    xt = _linear_relu(x, W, b)
    src = edge_index[0]
    dst = edge_index[1]
    parts, rs = _make_sc_agg(n, d, e)(xt, src, dst, edge_weight)
    blk = 1000
    rs3 = rs.reshape(NW, n // blk, blk).swapaxes(0, 1)
    return _combine(parts, rs3, n, d)


# speedup vs baseline: 11.9291x; 1.0601x over previous
import functools

import jax
import jax.numpy as jnp
from jax import lax
from jax.experimental import pallas as pl
from jax.experimental.pallas import tpu as pltpu
from jax.experimental.pallas import tpu_sc as plsc

NC = 2
NS = 16
NW = NC * NS
L = 16

CHUNK = 80
NSLOT = 3
NROT = 3


def _linear_relu(x, W, b):
    n, d = x.shape
    blk = 1000
    assert n % blk == 0

    def body(x_ref, w_ref, b_ref, o_ref):
        y = lax.dot_general(x_ref[...], w_ref[...], (((1,), (1,)), ((), ())),
                            preferred_element_type=jnp.float32)
        o_ref[...] = jnp.maximum(y + b_ref[...], 0.0)

    return pl.pallas_call(
        body,
        grid=(n // blk,),
        in_specs=[
            pl.BlockSpec((blk, d), lambda i: (i, 0)),
            pl.BlockSpec((d, d), lambda i: (0, 0)),
            pl.BlockSpec((1, d), lambda i: (0, 0)),
        ],
        out_specs=pl.BlockSpec((blk, d), lambda i: (i, 0)),
        out_shape=jax.ShapeDtypeStruct((n, d), jnp.float32),
    )(x, W, b.reshape(1, d))


def _make_sc_agg(n, d, e):
    ept = e // NW
    assert ept * NW == e and ept % CHUNK == 0
    steps = ept // CHUNK
    trips = (steps - 2) // NSLOT
    assert trips * NSLOT + 2 == steps
    span = -(-(n - 8 * CHUNK) // (NS - 1)) if NS > 1 else 0
    span -= span % 8
    zchunks = 8
    assert span * (NS - 1) + zchunks * CHUNK >= n and span > 0
    mesh = plsc.VectorSubcoreMesh(core_axis_name="c", subcore_axis_name="s",
                                  num_cores=NC, num_subcores=NS)

    @functools.partial(
        pl.kernel,
        out_type=(
            jax.ShapeDtypeStruct((NC, n, d), jnp.float32),
            jax.ShapeDtypeStruct((NW * n,), jnp.float32),
        ),
        mesh=mesh,
        scratch_types=[
            pltpu.VMEM_SHARED((n, d), jnp.float32),
            pltpu.VMEM((n,), jnp.float32),
            pltpu.VMEM((NSLOT, CHUNK), jnp.int32),
            pltpu.VMEM((NSLOT, CHUNK), jnp.int32),
            pltpu.VMEM((NSLOT, CHUNK), jnp.float32),
            pltpu.VMEM((NROT, CHUNK, d), jnp.float32),
            pltpu.SemaphoreType.DMA,
            pltpu.SemaphoreType.DMA,
            pltpu.SemaphoreType.DMA,
        ],
        compiler_params=pltpu.CompilerParams(needs_layout_passes=False),
    )
    def sc_agg(xt, src_h, dst_h, w_h, out, rs_out,
               aggr_sh, rs_v, dst_r, src_r, w_r, rows_v, gsem, ssem, isem):
        cid = lax.axis_index("c")
        sid = lax.axis_index("s")
        wid = sid * NC + cid

        zero16 = jnp.zeros((L,), jnp.float32)

        @plsc.parallel_loop(0, NROT * CHUNK * (d // L), unroll=8)
        def _zrow(i):
            rows_v[i // (d // L) // CHUNK, (i // (d // L)) % CHUNK,
                   pl.ds((i % (d // L)) * L, L)] = zero16

        @plsc.parallel_loop(0, n // L, unroll=8)
        def _zrs(i):
            rs_v[pl.ds(i * L, L)] = zero16

        row0 = pl.multiple_of(sid * span, 8)
        zdescs = [
            pltpu.async_copy(rows_v.at[0],
                             aggr_sh.at[pl.ds(row0 + k * CHUNK, CHUNK)], gsem)
            for k in range(zchunks)
        ]
        for zd in zdescs:
            zd.wait()
        plsc.subcore_barrier()

        ebase = pl.multiple_of(wid * ept, 8)

        def chunk_off(t):
            return pl.multiple_of(ebase + lax.rem(t, steps) * CHUNK, 8)

        def load_idx(t, slot):
            o = chunk_off(t)
            pltpu.async_copy(dst_h.at[pl.ds(o, CHUNK)], dst_r.at[slot], isem)
            pltpu.async_copy(src_h.at[pl.ds(o, CHUNK)], src_r.at[slot], isem)
            pltpu.async_copy(w_h.at[pl.ds(o, CHUNK)], w_r.at[slot], isem)

        def wait_idx(slot):
            for ref in (dst_r, src_r, w_r):
                pltpu.make_async_copy(dst_h.at[pl.ds(ebase, CHUNK)],
                                      ref.at[slot], isem).wait()

        def start_gather(slot, r):
            pltpu.async_copy(xt.at[dst_r.at[slot]], rows_v.at[r], gsem)

        def wait_gather(slot, r):
            pltpu.make_async_copy(xt.at[dst_r.at[slot]], rows_v.at[r],
                                  gsem).wait()

        def start_scatter(slot, r):
            pltpu.async_copy(rows_v.at[r], aggr_sh.at[src_r.at[slot]], ssem,
                             add=True)

        def wait_scatter(slot, r):
            pltpu.make_async_copy(rows_v.at[r], aggr_sh.at[src_r.at[slot]],
                                  ssem).wait()

        def scale_and_deg(slot, r):
            @plsc.parallel_loop(0, CHUNK, unroll=4)
            def _row_body(i):
                wb = plsc.load_gather(w_r.at[slot],
                                      [jnp.zeros((L,), jnp.int32) + i])
                for cb in range(d // L):
                    rows_v[r, i, pl.ds(cb * L, L)] = (
                        rows_v[r, i, pl.ds(cb * L, L)] * wb)

            for g in range(CHUNK // L):
                sv = src_r[slot, pl.ds(g * L, L)]
                wv = w_r[slot, pl.ds(g * L, L)]
                plsc.addupdate_scatter(rs_v, [sv], wv)

        load_idx(0, 0)
        for _ in range(3):
            pltpu.make_async_copy(dst_h.at[pl.ds(ebase, CHUNK)],
                                  dst_r.at[0], isem).wait()
        load_idx(1, 1)
        start_gather(0, 0)
        start_scatter(0, 1)
        start_scatter(0, 2)

        def trip_body(k, _):
            t0 = k * NSLOT
            for j in range(NSLOT):
                t = t0 + j
                sj = j
                sn = (j + 1) % NSLOT
                sp = (j + 2) % NSLOT
                wait_scatter(sn, sn)
                wait_idx(sn)
                start_gather(sn, sn)
                wait_gather(sj, sj)
                load_idx(t + 2, sp)
                scale_and_deg(sj, sj)
                start_scatter(sj, sj)
            return 0

        lax.fori_loop(0, trips, trip_body, 0)

        wait_scatter(1, 1)
        wait_idx(1)
        start_gather(1, 1)
        wait_gather(0, 0)
        scale_and_deg(0, 0)
        start_scatter(0, 0)
        wait_scatter(2, 2)
        wait_gather(1, 1)
        scale_and_deg(1, 1)
        start_scatter(1, 1)
        wait_scatter(0, 0)
        wait_scatter(1, 1)

        plsc.subcore_barrier()

        wdescs = [
            pltpu.async_copy(aggr_sh.at[pl.ds(pl.multiple_of(row0 + k * CHUNK, 8), CHUNK)],
                             out.at[cid, pl.ds(pl.multiple_of(row0 + k * CHUNK, 8), CHUNK)],
                             gsem)
            for k in range(zchunks)
        ]
        wdescs.append(pltpu.async_copy(rs_v, rs_out.at[pl.ds(wid * n, n)], ssem))
        for wd in wdescs:
            wd.wait()

    return sc_agg


def _combine(parts, rs, n, d):
    blk = 1000
    assert n % blk == 0

    def body(p_ref, r_ref, o_ref):
        s = p_ref[0] + p_ref[1]
        deg = lax.dot_general(r_ref[0], jnp.ones((NW, 1), jnp.float32),
                              (((0,), (0,)), ((), ())),
                              preferred_element_type=jnp.float32)
        o_ref[...] = s / jnp.maximum(deg, 1.0)

    return pl.pallas_call(
        body,
        grid=(n // blk,),
        in_specs=[
            pl.BlockSpec((NC, blk, d), lambda i: (0, i, 0)),
            pl.BlockSpec((1, NW, blk), lambda i: (i, 0, 0)),
        ],
        out_specs=pl.BlockSpec((blk, d), lambda i: (i, 0)),
        out_shape=jax.ShapeDtypeStruct((n, d), jnp.float32),
    )(parts, rs)


def kernel(x, edge_index, num_node, edge_weight, W, b):
    del num_node
    n, d = x.shape
    e = edge_index.shape[1]
    xt = _linear_relu(x, W, b)
    src = edge_index[0]
    dst = edge_index[1]
    parts, rs = _make_sc_agg(n, d, e)(xt, src, dst, edge_weight)
    blk = 1000
    rs3 = rs.reshape(NW, n // blk, blk).swapaxes(0, 1)
    return _combine(parts, rs3, n, d)
